# baseline (speedup 1.0000x reference)
import jax
import jax.numpy as jnp
from jax.experimental import pallas as pl
from jax.experimental.pallas import tpu as pltpu

_EPS = 1e-5
_PAIRS = 2


def _model_kernel(x_ref, w_ref, pk_ref, se_w_ref, sp_w_ref, te_w_ref,
                  tp_w_ref, out_ref, m2s_ref, m2t_ref, bias_ref, colt_ref,
                  y_ref):
    xb = x_ref[0].astype(jnp.float32)
    P, L, C2 = xb.shape
    C = C2 // 2
    Hp = w_ref.shape[1] // 2
    H = out_ref.shape[3]
    G = pl.num_programs(0)
    i = pl.program_id(0)
    two = lambda v: jnp.concatenate([v, v], axis=1)

    alpha = pk_ref[:, 0:128][:, :C]
    rev_w = two(pk_ref[:, 128:256][:, :C])
    rev_b = two(pk_ref[:, 256:384][:, :C])
    sp_b = pk_ref[:, 384:512][:, :C]
    tp_b = pk_ref[:, 512:640][:, :C]
    D = se_w_ref.shape[1]
    se_b = pk_ref[:, 640:640 + D]
    te_b = pk_ref[:, 640 + D:640 + 2 * D]
    ob = 640 + 2 * D
    stb = pk_ref[:, ob:ob + Hp]
    ttb = pk_ref[:, ob + Hp:ob + 2 * Hp]

    @pl.when((i == 0) | (i == G // 2))
    def _():
        M_s = jnp.dot(se_w_ref[:], sp_w_ref[:],
                      preferred_element_type=jnp.float32)
        M_t = jnp.dot(te_w_ref[:], tp_w_ref[:],
                      preferred_element_type=jnp.float32)
        z = jnp.zeros((C, C), jnp.float32)
        m2s_ref[:] = jnp.concatenate([jnp.concatenate([M_s, z], axis=1),
                                      jnp.concatenate([z, M_s], axis=1)],
                                     axis=0)
        m2t_ref[:] = jnp.concatenate([jnp.concatenate([M_t, z], axis=1),
                                      jnp.concatenate([z, M_t], axis=1)],
                                     axis=0)
        bs = jnp.dot(se_b, sp_w_ref[:],
                     preferred_element_type=jnp.float32) + sp_b
        bt = jnp.dot(te_b, tp_w_ref[:],
                     preferred_element_type=jnp.float32) + tp_b
        ones_l = jnp.ones((1, L), jnp.bfloat16)
        cols = jax.lax.dot_general(ones_l, w_ref[:], (((1,), (0,)), ((), ())),
                                   preferred_element_type=jnp.float32)
        col_s = cols[:, :Hp]
        col_t = cols[:, Hp:]
        colt_ref[:] = col_t
        U = jnp.concatenate([col_s, col_t, stb + ttb], axis=0)
        V = jnp.concatenate([two(bs), two(bt), jnp.ones((1, C2), jnp.float32)],
                            axis=0)
        bias_ref[:] = jax.lax.dot_general(
            U, V, (((0,), (0,)), ((), ())),
            preferred_element_type=jnp.float32)

    mean = jnp.mean(xb, axis=1, keepdims=True)
    var = jnp.mean((xb - mean) ** 2, axis=1, keepdims=True)
    stdev = jnp.sqrt(var + _EPS)
    s = rev_w / stdev
    o = rev_b - mean * s

    a = two(jax.nn.sigmoid(alpha))
    a3 = a[None]
    u = xb
    p = a
    d = 1
    while d < 64:
        shifted = jnp.concatenate(
            [jnp.zeros((P, d, C2), xb.dtype), u[:, :L - d]], axis=1)
        u = u + p * shifted
        p = p * p
        d *= 2
    y_ref[:] = u
    x0 = xb[:, 0:1, :]
    log_a = jnp.log(a3)
    tail = jnp.max(p) > 1e-10

    @pl.when(tail)
    def _():
        uu = y_ref[:]
        pp = p
        dd = 64
        while dd < L:
            sh = jnp.concatenate(
                [jnp.zeros((P, dd, C2), xb.dtype), uu[:, :L - dd]], axis=1)
            uu = uu + pp * sh
            pp = pp * pp
            dd *= 2
        io = jax.lax.broadcasted_iota(
            jnp.int32, (1, L, C2), 1).astype(jnp.float32)
        y_ref[:] = (1.0 - a3) * uu + x0 * jnp.exp(log_a * (io + 1.0))

    @pl.when(jnp.logical_not(tail))
    def _():
        io = jax.lax.broadcasted_iota(
            jnp.int32, (1, 64, C2), 1).astype(jnp.float32)
        y_ref[:] = (1.0 - a3) * y_ref[:]
        y_ref[:, :64, :] = (y_ref[:, :64, :]
                            + x0 * jnp.exp(log_a * (io + 1.0)))

    tr = y_ref[:]
    diff = xb - tr

    dn0 = (((0,), (0,)), ((), ()))
    dn_std = (((1,), (0,)), ((), ()))
    inv_rw = 1.0 / (rev_w + _EPS)
    for j in range(P):
        sj = s[j]
        Sr = jax.lax.dot_general(w_ref[:, :Hp], diff[j].astype(jnp.bfloat16),
                                 dn0, preferred_element_type=jnp.float32)
        Tr = jax.lax.dot_general(w_ref[:, Hp:], tr[j].astype(jnp.bfloat16),
                                 dn0, preferred_element_type=jnp.float32)
        oM = jax.lax.dot_general(o[j], m2t_ref[:], dn_std,
                                 preferred_element_type=jnp.float32)
        outp = jax.lax.dot_general(Sr * sj, m2s_ref[:], dn_std,
                                   preferred_element_type=jnp.float32) \
            + jax.lax.dot_general(Tr * sj, m2t_ref[:], dn_std,
                                  preferred_element_type=jnp.float32) \
            + jax.lax.dot_general(colt_ref[:], oM, dn0,
                                  preferred_element_type=jnp.float32) \
            + bias_ref[:]
        q = inv_rw * stdev[j]
        r = mean[j] - rev_b * q
        out = outp * q + r
        out_ref[0, j, 0] = out[:H, :C]
        out_ref[0, j, 1] = out[:H, C:]


def kernel(x, alpha, rev_w, rev_b, se_w, se_b, sp_w, sp_b, st_w, st_b,
           te_w, te_b, tp_w, tp_b, tt_w, tt_b, interpret=False):
    B, L, C = x.shape
    H = st_w.shape[1]
    D = se_w.shape[1]
    P = _PAIRS
    G = B // (2 * P)
    Hp = (H + 127) // 128 * 128

    xp = x.reshape(G, P, 2, L, C).transpose(0, 1, 3, 2, 4).reshape(
        G, P, L, 2 * C).astype(jnp.bfloat16)
    zH = jnp.zeros((L, Hp - H), jnp.bfloat16)
    w_cat = jnp.concatenate([st_w.astype(jnp.bfloat16), zH,
                             tt_w.astype(jnp.bfloat16), zH], axis=1)
    zC = jnp.zeros((128 - C,), jnp.float32)
    zHb = jnp.zeros((Hp - H,), jnp.float32)
    pk = jnp.concatenate([
        alpha, zC, rev_w, zC, rev_b, zC, sp_b, zC, tp_b, zC,
        se_b, te_b, st_b, zHb, tt_b, zHb]).reshape(1, -1)

    full = lambda s: pl.BlockSpec(s, lambda b: (0,) * len(s))

    out2 = pl.pallas_call(
        _model_kernel,
        grid=(G,),
        in_specs=[
            pl.BlockSpec((1, P, L, 2 * C), lambda b: (b, 0, 0, 0)),
            full((L, 2 * Hp)),
            full((1, 640 + 2 * D + 2 * Hp)),
            full((C, D)), full((D, C)), full((C, D)), full((D, C)),
        ],
        out_specs=pl.BlockSpec((1, P, 2, H, C), lambda b: (b, 0, 0, 0, 0)),
        out_shape=jax.ShapeDtypeStruct((G, P, 2, H, C), jnp.float32),
        scratch_shapes=[
            pltpu.VMEM((2 * C, 2 * C), jnp.float32),
            pltpu.VMEM((2 * C, 2 * C), jnp.float32),
            pltpu.VMEM((Hp, 2 * C), jnp.float32),
            pltpu.VMEM((1, Hp), jnp.float32),
            pltpu.VMEM((P, L, 2 * C), jnp.float32),
        ],
        compiler_params=pltpu.CompilerParams(
            dimension_semantics=("parallel",),
            vmem_limit_bytes=56 * 1024 * 1024,
        ),
        name="starlivts_fused",
        interpret=interpret,
    )(xp, w_cat, pk, se_w, sp_w, te_w, tp_w)

    return out2.reshape(B, H, C)


# speedup vs baseline: 1.0687x; 1.0687x over previous
import jax
import jax.numpy as jnp
from jax.experimental import pallas as pl
from jax.experimental.pallas import tpu as pltpu

_EPS = 1e-5
_PAIRS = 2


def _model_kernel(x_ref, alpha_ref, rev_w_ref, rev_b_ref,
                  se_w_ref, se_b_ref, sp_w_ref, sp_b_ref, st_w_ref, st_b_ref,
                  te_w_ref, te_b_ref, tp_w_ref, tp_b_ref, tt_w_ref, tt_b_ref,
                  out_ref, m2s_ref, m2t_ref, bias_ref, colt_ref, y_ref):
    xb = x_ref[0]
    P, L, C2 = xb.shape
    C = C2 // 2
    G = pl.num_programs(0)
    i = pl.program_id(0)
    two = lambda v: jnp.concatenate([v, v], axis=1)
    rev_w = two(rev_w_ref[:])
    rev_b = two(rev_b_ref[:])

    @pl.when((i == 0) | (i == G // 2))
    def _():
        M_s = jnp.dot(se_w_ref[:], sp_w_ref[:],
                      preferred_element_type=jnp.float32)
        M_t = jnp.dot(te_w_ref[:], tp_w_ref[:],
                      preferred_element_type=jnp.float32)
        z = jnp.zeros((C, C), jnp.float32)
        m2s_ref[:] = jnp.concatenate([jnp.concatenate([M_s, z], axis=1),
                                      jnp.concatenate([z, M_s], axis=1)],
                                     axis=0)
        m2t_ref[:] = jnp.concatenate([jnp.concatenate([M_t, z], axis=1),
                                      jnp.concatenate([z, M_t], axis=1)],
                                     axis=0)
        bs = jnp.dot(se_b_ref[:], sp_w_ref[:],
                     preferred_element_type=jnp.float32) + sp_b_ref[:]
        bt = jnp.dot(te_b_ref[:], tp_w_ref[:],
                     preferred_element_type=jnp.float32) + tp_b_ref[:]
        ones_l = jnp.ones((1, L), jnp.bfloat16)
        dn_lk = (((1,), (0,)), ((), ()))
        col_s = jax.lax.dot_general(ones_l, st_w_ref[:], dn_lk,
                                    preferred_element_type=jnp.float32)
        col_t = jax.lax.dot_general(ones_l, tt_w_ref[:], dn_lk,
                                    preferred_element_type=jnp.float32)
        colt_ref[:] = col_t
        U = jnp.concatenate([col_s, col_t, st_b_ref[:] + tt_b_ref[:]], axis=0)
        V = jnp.concatenate([two(bs), two(bt), jnp.ones((1, C2), jnp.float32)],
                            axis=0)
        bias_ref[:] = jax.lax.dot_general(
            U, V, (((0,), (0,)), ((), ())),
            preferred_element_type=jnp.float32)

    mean = jnp.mean(xb, axis=1, keepdims=True)
    var = jnp.mean((xb - mean) ** 2, axis=1, keepdims=True)
    stdev = jnp.sqrt(var + _EPS)
    s = rev_w / stdev
    o = rev_b - mean * s

    a = two(jax.nn.sigmoid(alpha_ref[:]))
    row_is0 = jax.lax.broadcasted_iota(jnp.int32, (1, L, 1), 1) == 0
    y = jnp.where(row_is0, xb, (1.0 - a) * xb)
    p = a
    d = 1
    while d < 64:
        shifted = jnp.concatenate(
            [jnp.zeros((P, d, C2), xb.dtype), y[:, :L - d]], axis=1)
        y = y + p * shifted
        p = p * p
        d *= 2
    y_ref[:] = y

    @pl.when(jnp.max(p) > 1e-10)
    def _():
        yy = y_ref[:]
        pp = p
        dd = 64
        while dd < L:
            sh = jnp.concatenate(
                [jnp.zeros((P, dd, C2), xb.dtype), yy[:, :L - dd]], axis=1)
            yy = yy + pp * sh
            pp = pp * pp
            dd *= 2
        y_ref[:] = yy

    tr = y_ref[:]
    diff = xb - tr

    dn0 = (((0,), (0,)), ((), ()))
    dn_std = (((1,), (0,)), ((), ()))
    inv_rw = 1.0 / (rev_w + _EPS)
    for j in range(P):
        sj = s[j]
        Sr = jax.lax.dot_general(st_w_ref[:], diff[j].astype(jnp.bfloat16),
                                 dn0, preferred_element_type=jnp.float32)
        Tr = jax.lax.dot_general(tt_w_ref[:], tr[j].astype(jnp.bfloat16),
                                 dn0, preferred_element_type=jnp.float32)
        oM = jax.lax.dot_general(o[j], m2t_ref[:], dn_std,
                                 preferred_element_type=jnp.float32)
        outp = jax.lax.dot_general(Sr * sj, m2s_ref[:], dn_std,
                                   preferred_element_type=jnp.float32) \
            + jax.lax.dot_general(Tr * sj, m2t_ref[:], dn_std,
                                  preferred_element_type=jnp.float32) \
            + jax.lax.dot_general(colt_ref[:], oM, dn0,
                                  preferred_element_type=jnp.float32) \
            + bias_ref[:]
        q = inv_rw * stdev[j]
        r = mean[j] - rev_b * q
        out = outp * q + r
        out_ref[0, j, 0] = out[:, :C]
        out_ref[0, j, 1] = out[:, C:]


def kernel(x, alpha, rev_w, rev_b, se_w, se_b, sp_w, sp_b, st_w, st_b,
           te_w, te_b, tp_w, tp_b, tt_w, tt_b, interpret=False):
    B, L, C = x.shape
    H = st_w.shape[1]
    D = se_w.shape[1]
    P = _PAIRS
    G = B // (2 * P)

    xp = x.reshape(G, P, 2, L, C).transpose(0, 1, 3, 2, 4).reshape(
        G, P, L, 2 * C)
    st_bf = st_w.astype(jnp.bfloat16)
    tt_bf = tt_w.astype(jnp.bfloat16)

    vec = lambda v: v.reshape(1, -1)
    full = lambda s: pl.BlockSpec(s, lambda b: (0,) * len(s))

    out2 = pl.pallas_call(
        _model_kernel,
        grid=(G,),
        in_specs=[
            pl.BlockSpec((1, P, L, 2 * C), lambda b: (b, 0, 0, 0)),
            full((1, C)), full((1, C)), full((1, C)),
            full((C, D)), full((1, D)), full((D, C)), full((1, C)),
            full((L, H)), full((1, H)),
            full((C, D)), full((1, D)), full((D, C)), full((1, C)),
            full((L, H)), full((1, H)),
        ],
        out_specs=pl.BlockSpec((1, P, 2, H, C), lambda b: (b, 0, 0, 0, 0)),
        out_shape=jax.ShapeDtypeStruct((G, P, 2, H, C), jnp.float32),
        scratch_shapes=[
            pltpu.VMEM((2 * C, 2 * C), jnp.float32),
            pltpu.VMEM((2 * C, 2 * C), jnp.float32),
            pltpu.VMEM((H, 2 * C), jnp.float32),
            pltpu.VMEM((1, H), jnp.float32),
            pltpu.VMEM((P, L, 2 * C), jnp.float32),
        ],
        compiler_params=pltpu.CompilerParams(
            dimension_semantics=("parallel",),
            vmem_limit_bytes=56 * 1024 * 1024,
        ),
        name="starlivts_fused",
        interpret=interpret,
    )(xp, vec(alpha), vec(rev_w), vec(rev_b),
      se_w, vec(se_b), sp_w, vec(sp_b), st_bf, vec(st_b),
      te_w, vec(te_b), tp_w, vec(tp_b), tt_bf, vec(tt_b))

    return out2.reshape(B, H, C)
